# baseline (speedup 1.0000x reference)
import functools

import jax
import jax.numpy as jnp
from jax import lax
from jax.experimental import pallas as pl
from jax.experimental.pallas import tpu as pltpu
from jax.experimental.pallas import tpu_sc as plsc

NUM_NODES = 32768
TABLE_ROWS = 65536
CH = 4
B = 128

_info = plsc.get_sparse_core_info()
NC, NS, L = _info.num_cores, _info.num_subcores, _info.num_lanes
NW = NC * NS
HALF = NUM_NODES // NC
NODES_PER_W = NUM_NODES // NW
GN = 32
IDX = GN * CH
STEPS = NODES_PER_W // GN
NBUF = 4


_mesh = plsc.VectorSubcoreMesh(core_axis_name="c", subcore_axis_name="s")


@functools.partial(
    pl.kernel,
    mesh=_mesh,
    out_type=[jax.ShapeDtypeStruct((HALF, B), jnp.float32),
              jax.ShapeDtypeStruct((HALF, B), jnp.float32)],
    scratch_types=(
        [pltpu.VMEM((NODES_PER_W * CH,), jnp.int32)]
        + [pltpu.VMEM((IDX, B), jnp.float32)] * NBUF
        + [pltpu.VMEM((GN, B), jnp.float32)] * NBUF
        + [pltpu.SemaphoreType.DMA] * (2 * NBUF)
    ),
)
def _prod_fwd(node_hbm, cids_hbm, outa_hbm, outb_hbm, idx_all, *bufs):
    rows = bufs[0:NBUF]
    outs = bufs[NBUF:2 * NBUF]
    gsems = bufs[2 * NBUF:3 * NBUF]
    ssems = bufs[3 * NBUF:4 * NBUF]

    cid = lax.axis_index("c")
    sid = lax.axis_index("s")
    lbase = sid * NODES_PER_W
    base = cid * HALF + lbase

    pltpu.sync_copy(cids_hbm.at[pl.ds(base * CH, NODES_PER_W * CH)], idx_all)

    def gather(g, b):
        return pltpu.async_copy(
            node_hbm.at[idx_all.at[pl.ds(g * IDX, IDX)]], rows[b], gsems[b])

    for b in range(NBUF - 1):
        gather(b, b)

    def ring(t, carry):
        for b in range(NBUF):
            g = NBUF * t + b
            pltpu.make_async_copy(
                node_hbm.at[idx_all.at[pl.ds(g * IDX, IDX)]],
                rows[b], gsems[b]).wait()

            nb = (b + NBUF - 1) % NBUF

            @pl.when(g + NBUF - 1 < STEPS)
            def _():
                gather(g + NBUF - 1, nb)

            n0 = lbase + g * GN
            out_v, rows_v = outs[b], rows[b]

            @pl.when(t > 0)
            def _():
                @pl.when(cid == 0)
                def _():
                    pltpu.make_async_copy(
                        out_v, outa_hbm.at[pl.ds(0, GN)], ssems[b]).wait()

                @pl.when(cid == 1)
                def _():
                    pltpu.make_async_copy(
                        out_v, outb_hbm.at[pl.ds(0, GN)], ssems[b]).wait()

            @plsc.parallel_loop(0, GN, 1, unroll=4)
            def node_body(j):
                r = CH * j
                for v in range(B // L):
                    s = pl.ds(v * L, L)
                    out_v[j, s] = (rows_v[r, s] + rows_v[r + 1, s]
                                   + rows_v[r + 2, s] + rows_v[r + 3, s])

            @pl.when(cid == 0)
            def _():
                pltpu.async_copy(out_v, outa_hbm.at[pl.ds(n0, GN)], ssems[b])

            @pl.when(cid == 1)
            def _():
                pltpu.async_copy(out_v, outb_hbm.at[pl.ds(n0, GN)], ssems[b])
        return carry

    lax.fori_loop(0, STEPS // NBUF, ring, 0)

    for b in range(NBUF):
        @pl.when(cid == 0)
        def _():
            pltpu.make_async_copy(
                outs[b], outa_hbm.at[pl.ds(0, GN)], ssems[b]).wait()

        @pl.when(cid == 1)
        def _():
            pltpu.make_async_copy(
                outs[b], outb_hbm.at[pl.ds(0, GN)], ssems[b]).wait()


def kernel(node_mars, element_mars, cids):
    outa, outb = _prod_fwd(node_mars, cids.reshape(-1))
    return jnp.concatenate([element_mars[0:1, :], outa, outb], axis=0)


# speedup vs baseline: 1.5441x; 1.5441x over previous
import functools

import jax
import jax.numpy as jnp
from jax import lax
from jax.experimental import pallas as pl
from jax.experimental.pallas import tpu as pltpu
from jax.experimental.pallas import tpu_sc as plsc

NUM_NODES = 32768
TABLE_ROWS = 65536
CH = 4
B = 128

_info = plsc.get_sparse_core_info()
NC, NS, L = _info.num_cores, _info.num_subcores, _info.num_lanes
NW = NC * NS
NODES_PER_W = NUM_NODES // NW
GN = 32
IDX = GN * CH
STEPS = NODES_PER_W // GN
NBUF = 4


_mesh = plsc.VectorSubcoreMesh(core_axis_name="c", subcore_axis_name="s")


@functools.partial(
    pl.kernel,
    mesh=_mesh,
    out_type=jax.ShapeDtypeStruct((NUM_NODES + 1, B), jnp.float32),
    scratch_types=(
        [pltpu.VMEM((NODES_PER_W * CH,), jnp.int32)]
        + [pltpu.VMEM((IDX, B), jnp.float32)] * NBUF
        + [pltpu.VMEM((GN, B), jnp.float32)] * NBUF
        + [pltpu.VMEM((GN,), jnp.int32)] * NBUF
        + [pltpu.SemaphoreType.DMA] * (2 * NBUF)
    ),
)
def _prod_fwd(node_hbm, cids_hbm, out_hbm, idx_all, *bufs):
    rows = bufs[0:NBUF]
    outs = bufs[NBUF:2 * NBUF]
    oidxs = bufs[2 * NBUF:3 * NBUF]
    gsems = bufs[3 * NBUF:4 * NBUF]
    ssems = bufs[4 * NBUF:5 * NBUF]

    wid = lax.axis_index("s") * NC + lax.axis_index("c")
    base = wid * NODES_PER_W
    iota = lax.broadcasted_iota(jnp.int32, (L,), 0)

    pltpu.sync_copy(cids_hbm.at[pl.ds(base * CH, NODES_PER_W * CH)], idx_all)

    def gather(g, b):
        return pltpu.async_copy(
            node_hbm.at[idx_all.at[pl.ds(g * IDX, IDX)]], rows[b], gsems[b])

    for b in range(NBUF - 1):
        gather(b, b)

    def ring(t, carry):
        for b in range(NBUF):
            g = NBUF * t + b
            pltpu.make_async_copy(
                node_hbm.at[idx_all.at[pl.ds(g * IDX, IDX)]],
                rows[b], gsems[b]).wait()

            nb = (b + NBUF - 1) % NBUF

            @pl.when(g + NBUF - 1 < STEPS)
            def _():
                gather(g + NBUF - 1, nb)

            @pl.when(t > 0)
            def _():
                pltpu.make_async_copy(
                    outs[b], out_hbm.at[oidxs[b]], ssems[b]).wait()

            n0 = base + g * GN
            out_v, oidx_v, rows_v = outs[b], oidxs[b], rows[b]
            for v in range(GN // L):
                oidx_v[pl.ds(v * L, L)] = iota + (1 + n0 + v * L)

            @plsc.parallel_loop(0, GN, 1, unroll=4)
            def node_body(j):
                r = CH * j
                for v in range(B // L):
                    s = pl.ds(v * L, L)
                    out_v[j, s] = (rows_v[r, s] + rows_v[r + 1, s]
                                   + rows_v[r + 2, s] + rows_v[r + 3, s])

            pltpu.async_copy(out_v, out_hbm.at[oidx_v], ssems[b])
        return carry

    lax.fori_loop(0, STEPS // NBUF, ring, 0)

    for b in range(NBUF):
        pltpu.make_async_copy(outs[b], out_hbm.at[oidxs[b]], ssems[b]).wait()


def kernel(node_mars, element_mars, cids):
    out = _prod_fwd(node_mars, cids.reshape(-1))
    return out.at[0:1, :].set(element_mars[0:1, :])
